# Initial kernel scaffold; baseline (speedup 1.0000x reference)
import jax
import jax.numpy as jnp
from jax.experimental import pallas as pl


def kernel(x, edge_index, edge_norm, edge_type, W_rel, W_self, b, W_out, b_out):
    raise NotImplementedError("write your pallas kernel here")


# speedup vs baseline: 9.3590x; 9.3590x over previous
import jax
import jax.numpy as jnp
from jax import lax
from jax.experimental import pallas as pl
from jax.experimental.pallas import tpu as pltpu
from jax.experimental.pallas import tpu_sc as plsc

N = 10000
E = 320000
D = 200
H = 80
R = 8
T = 6

NC = 2
NS = 16
NW = NC * NS
EPW = E // NW
CH = 80
NCHUNK = EPW // CH
NPAD = 10240
ROWS_PT = NPAD // NS
LANES = 16


def _rel_transform_body(x_ref, wrel_ref, wself_ref, hrel_ref, xw_ref):
    xb = x_ref[...]
    for r in range(R):
        hrel_ref[r] = jnp.dot(xb, wrel_ref[r], preferred_element_type=jnp.float32)
    xw_ref[...] = jnp.dot(xb, wself_ref[...], preferred_element_type=jnp.float32)


def _edge_agg_body(hrel_hbm, src_hbm, dst_hbm, typ_hbm, nrm_hbm, out_hbm,
                   src_v, dst_v, typ_v, nrm_v, idx_v, rows_v, zero_v,
                   acc_sh, sem):
    c = lax.axis_index("c")
    s = lax.axis_index("s")
    wid = s * NC + c

    zvec = jnp.zeros((LANES,), jnp.float32)

    def zrow(j, carry):
        for k in range(H // LANES):
            zero_v[j, pl.ds(k * LANES, LANES)] = zvec
        return carry

    lax.fori_loop(0, CH, zrow, 0)
    row0 = s * ROWS_PT
    for k in range(ROWS_PT // CH):
        pltpu.sync_copy(zero_v.at[pl.ds(0, CH)],
                        acc_sh.at[pl.ds(row0 + k * CH, CH)])
    plsc.subcore_barrier()

    base_edge = wid * EPW

    def chunk(i, carry):
        off = pl.multiple_of(base_edge + i * CH, 8)
        pltpu.sync_copy(src_hbm.at[pl.ds(off, CH)], src_v)
        pltpu.sync_copy(typ_hbm.at[pl.ds(off, CH)], typ_v)
        pltpu.sync_copy(nrm_hbm.at[pl.ds(off, CH)], nrm_v)
        pltpu.sync_copy(dst_hbm.at[pl.ds(off, CH)], dst_v)
        for j in range(CH // LANES):
            sl = pl.ds(j * LANES, LANES)
            idx_v[sl] = typ_v[sl] * N + src_v[sl]
        pltpu.async_copy(hrel_hbm.at[idx_v], rows_v, sem).wait()

        lane = lax.iota(jnp.int32, LANES)

        def scale_group(j, carry2):
            jb = pl.multiple_of(j * LANES, 8)
            nv = nrm_v[pl.ds(jb, LANES)]
            for t in range(LANES):
                e = jb + t
                spl = jnp.full((LANES,),
                               jnp.sum(jnp.where(lane == t, nv, 0.0)),
                               jnp.float32)
                for k in range(H // LANES):
                    sl = pl.ds(k * LANES, LANES)
                    rows_v[e, sl] = rows_v[e, sl] * spl
            return carry2

        lax.fori_loop(0, CH // LANES, scale_group, 0)
        pltpu.sync_copy(rows_v, acc_sh.at[dst_v], add=True)
        return carry

    lax.fori_loop(0, NCHUNK, chunk, 0)

    plsc.subcore_barrier()
    pltpu.sync_copy(acc_sh.at[pl.ds(row0, ROWS_PT)],
                    out_hbm.at[c, pl.ds(row0, ROWS_PT)])


def _finish_body(p_ref, xw_ref, b_ref, wout_ref, bout_ref, out_ref):
    h = p_ref[0] + p_ref[1] + xw_ref[...] + b_ref[...]
    h = jnp.maximum(h, 0.0)
    logits = jnp.dot(h, wout_ref[...], preferred_element_type=jnp.float32)
    logits = logits + bout_ref[...]
    m = jnp.max(logits, axis=1, keepdims=True)
    lse = jnp.log(jnp.sum(jnp.exp(logits - m), axis=1, keepdims=True)) + m
    out_ref[...] = logits - lse


def kernel(x, edge_index, edge_norm, edge_type, W_rel, W_self, b, W_out, b_out):
    src = edge_index[0].astype(jnp.int32)
    dst = edge_index[1].astype(jnp.int32)
    typ = edge_type.astype(jnp.int32)
    nrm = edge_norm.astype(jnp.float32)

    BN = 1000
    hrel, xw = pl.pallas_call(
        _rel_transform_body,
        grid=(N // BN,),
        in_specs=[pl.BlockSpec((BN, D), lambda i: (i, 0)),
                  pl.BlockSpec((R, D, H), lambda i: (0, 0, 0)),
                  pl.BlockSpec((D, H), lambda i: (0, 0))],
        out_specs=[pl.BlockSpec((R, BN, H), lambda i: (0, i, 0)),
                   pl.BlockSpec((BN, H), lambda i: (i, 0))],
        out_shape=[jax.ShapeDtypeStruct((R, N, H), jnp.float32),
                   jax.ShapeDtypeStruct((N, H), jnp.float32)],
    )(x, W_rel, W_self)
    hrel_flat = hrel.reshape(R * N, H)

    mesh = plsc.VectorSubcoreMesh(core_axis_name="c", subcore_axis_name="s",
                                  num_cores=NC, num_subcores=NS)
    agg2 = pl.kernel(
        _edge_agg_body,
        out_type=jax.ShapeDtypeStruct((NC, NPAD, H), jnp.float32),
        mesh=mesh,
        compiler_params=pltpu.CompilerParams(use_tc_tiling_on_sc=False,
                                             needs_layout_passes=False),
        scratch_types=[
            pltpu.VMEM((CH,), jnp.int32),
            pltpu.VMEM((CH,), jnp.int32),
            pltpu.VMEM((CH,), jnp.int32),
            pltpu.VMEM((CH,), jnp.float32),
            pltpu.VMEM((CH,), jnp.int32),
            pltpu.VMEM((CH, H), jnp.float32),
            pltpu.VMEM((CH, H), jnp.float32),
            pltpu.VMEM_SHARED((NPAD, H), jnp.float32),
            pltpu.SemaphoreType.DMA,
        ],
    )(hrel_flat, src, dst, typ, nrm)

    out = pl.pallas_call(
        _finish_body,
        grid=(N // BN,),
        in_specs=[pl.BlockSpec((NC, BN, H), lambda i: (0, i, 0)),
                  pl.BlockSpec((BN, H), lambda i: (i, 0)),
                  pl.BlockSpec((1, H), lambda i: (0, 0)),
                  pl.BlockSpec((H, T), lambda i: (0, 0)),
                  pl.BlockSpec((1, T), lambda i: (0, 0))],
        out_specs=pl.BlockSpec((BN, T), lambda i: (i, 0)),
        out_shape=jax.ShapeDtypeStruct((N, T), jnp.float32),
    )(agg2, xw, b.reshape(1, H), W_out, b_out.reshape(1, T))
    return out
